# baseline (speedup 1.0000x reference)
import jax
import jax.numpy as jnp
from jax.experimental import pallas as pl
from jax.experimental.pallas import tpu as pltpu

_B = 16
_C = 64
_HW = 1024
_K = 1024
_NUMEL = _B * _C * _HW
_NTOK = _B * _HW


def _vq_body(x_ref, w_ref, idx_ref, q_ref, loss_ref, perp_ref, counts_ref,
             acc_ref):
    b = pl.program_id(0)
    X = x_ref[0]
    W = w_ref[...]
    M = jnp.dot(W.astype(jnp.bfloat16), X.astype(jnp.bfloat16),
                preferred_element_type=jnp.float32)
    xn = jnp.sum(X * X, axis=0)
    wn = jnp.sum(W * W, axis=1)
    T = (xn[None, :] + wn[:, None]) - 2.0 * M
    iota_k = jax.lax.broadcasted_iota(jnp.int32, (_K, _HW), 0)
    minv = jnp.min(T, axis=0)
    idx = jnp.min(jnp.where(T == minv[None, :], iota_k, _K), axis=0)
    E = (iota_k == idx[None, :]).astype(jnp.float32)
    Q = jax.lax.dot_general(W, E, (((0,), (0,)), ((), ())),
                            precision=jax.lax.Precision.HIGHEST,
                            preferred_element_type=jnp.float32)
    idx_ref[0, 0, :] = idx
    q_ref[0] = Q

    @pl.when(b == 0)
    def _init():
        counts_ref[...] = jnp.zeros_like(counts_ref)
        acc_ref[0] = 0.0

    counts_ref[...] += jnp.sum(E, axis=1)
    acc_ref[0] += jnp.sum(minv)

    @pl.when(b == _B - 1)
    def _finalize():
        loss_ref[...] = jnp.full((1, 1), acc_ref[0] * (1.25 / _NUMEL),
                                 jnp.float32)
        p = counts_ref[...] * (1.0 / _NTOK)
        perp_ref[...] = jnp.full((1, 1),
                                 jnp.exp(-jnp.sum(p * jnp.log(p + 1e-10))),
                                 jnp.float32)


def kernel(inputs, W):
    x3 = inputs.reshape(_B, _C, _HW)
    idx3, q3, loss11, perp11 = pl.pallas_call(
        _vq_body,
        grid=(_B,),
        in_specs=[pl.BlockSpec((1, _C, _HW), lambda b: (b, 0, 0)),
                  pl.BlockSpec((_K, _C), lambda b: (0, 0))],
        out_specs=[pl.BlockSpec((1, 1, _HW), lambda b: (b, 0, 0)),
                   pl.BlockSpec((1, _C, _HW), lambda b: (b, 0, 0)),
                   pl.BlockSpec((1, 1), lambda b: (0, 0)),
                   pl.BlockSpec((1, 1), lambda b: (0, 0))],
        out_shape=[jax.ShapeDtypeStruct((_B, 1, _HW), jnp.int32),
                   jax.ShapeDtypeStruct((_B, _C, _HW), jnp.float32),
                   jax.ShapeDtypeStruct((1, 1), jnp.float32),
                   jax.ShapeDtypeStruct((1, 1), jnp.float32)],
        scratch_shapes=[pltpu.VMEM((_K,), jnp.float32),
                        pltpu.SMEM((1,), jnp.float32)],
    )(x3, W)
    loss = loss11[0, 0]
    perplexity = perp11[0, 0]
    quantized_out = q3.reshape(inputs.shape)
    codebook_indices = idx3.reshape(-1)
    return (loss, quantized_out, perplexity, codebook_indices)


# speedup vs baseline: 1.3372x; 1.3372x over previous
import jax
import jax.numpy as jnp
from jax.experimental import pallas as pl
from jax.experimental.pallas import tpu as pltpu

_B = 16
_C = 64
_HW = 1024
_K = 1024
_NUMEL = _B * _C * _HW
_NTOK = _B * _HW


def _vq_body(x_ref, w_ref, idx_ref, q_ref, loss_ref, perp_ref, counts_ref,
             acc_ref):
    b = pl.program_id(0)
    X = x_ref[0]
    W = w_ref[...]
    W16 = W.astype(jnp.bfloat16)
    M = jnp.dot(W16, X.astype(jnp.bfloat16),
                preferred_element_type=jnp.float32)
    xn = jnp.sum(X * X, axis=0)
    wn = jnp.sum(W * W, axis=1)
    T = (xn[None, :] + wn[:, None]) - 2.0 * M
    iota_k = jax.lax.broadcasted_iota(jnp.int32, (_K, _HW), 0)
    minv = jnp.min(T, axis=0)
    idx = jnp.min(jnp.where(T == minv[None, :], iota_k, _K), axis=0)
    E16 = (iota_k == idx[None, :]).astype(jnp.bfloat16)
    Q = jax.lax.dot_general(W16, E16, (((0,), (0,)), ((), ())),
                            preferred_element_type=jnp.float32)
    idx_ref[0, 0, :] = idx
    q_ref[0] = Q

    @pl.when(b == 0)
    def _init():
        counts_ref[...] = jnp.zeros_like(counts_ref)
        acc_ref[0] = 0.0

    counts_ref[...] += jax.lax.dot_general(
        E16, jnp.ones((_HW,), jnp.bfloat16), (((1,), (0,)), ((), ())),
        preferred_element_type=jnp.float32)
    acc_ref[0] += jnp.sum(minv)

    @pl.when(b == _B - 1)
    def _finalize():
        loss_ref[...] = jnp.full((1, 1), acc_ref[0] * (1.25 / _NUMEL),
                                 jnp.float32)
        p = counts_ref[...] * (1.0 / _NTOK)
        perp_ref[...] = jnp.full((1, 1),
                                 jnp.exp(-jnp.sum(p * jnp.log(p + 1e-10))),
                                 jnp.float32)


def kernel(inputs, W):
    x3 = inputs.reshape(_B, _C, _HW)
    idx3, q3, loss11, perp11 = pl.pallas_call(
        _vq_body,
        grid=(_B,),
        in_specs=[pl.BlockSpec((1, _C, _HW), lambda b: (b, 0, 0)),
                  pl.BlockSpec((_K, _C), lambda b: (0, 0))],
        out_specs=[pl.BlockSpec((1, 1, _HW), lambda b: (b, 0, 0)),
                   pl.BlockSpec((1, _C, _HW), lambda b: (b, 0, 0)),
                   pl.BlockSpec((1, 1), lambda b: (0, 0)),
                   pl.BlockSpec((1, 1), lambda b: (0, 0))],
        out_shape=[jax.ShapeDtypeStruct((_B, 1, _HW), jnp.int32),
                   jax.ShapeDtypeStruct((_B, _C, _HW), jnp.float32),
                   jax.ShapeDtypeStruct((1, 1), jnp.float32),
                   jax.ShapeDtypeStruct((1, 1), jnp.float32)],
        scratch_shapes=[pltpu.VMEM((_K,), jnp.float32),
                        pltpu.SMEM((1,), jnp.float32)],
    )(x3, W)
    loss = loss11[0, 0]
    perplexity = perp11[0, 0]
    quantized_out = q3.reshape(inputs.shape)
    codebook_indices = idx3.reshape(-1)
    return (loss, quantized_out, perplexity, codebook_indices)


# speedup vs baseline: 1.6926x; 1.2658x over previous
import jax
import jax.numpy as jnp
from jax.experimental import pallas as pl
from jax.experimental.pallas import tpu as pltpu

_B = 16
_C = 64
_HW = 1024
_K = 1024
_NUMEL = _B * _C * _HW
_NTOK = _B * _HW


def _vq_body(x_ref, w_ref, idx_ref, q_ref, loss_ref, perp_ref, counts_ref,
             acc_ref):
    b = pl.program_id(0)
    X = x_ref[0]
    W = w_ref[...]
    W16 = W.astype(jnp.bfloat16)
    M = jnp.dot(W16, X.astype(jnp.bfloat16),
                preferred_element_type=jnp.float32)
    xn = jnp.sum(X * X, axis=0)
    wn = jnp.sum(W * W, axis=1)
    T = (xn[None, :] + wn[:, None]) - 2.0 * M
    iota_kf = jax.lax.broadcasted_iota(
        jnp.int32, (_K, _HW), 0).astype(jnp.float32)
    minv = jnp.min(T, axis=0)
    idxf = jnp.min(jnp.where(T == minv[None, :], iota_kf, float(_K)), axis=0)
    idx = idxf.astype(jnp.int32)
    E16 = (iota_kf == idxf[None, :]).astype(jnp.bfloat16)
    Q = jax.lax.dot_general(W16, E16, (((0,), (0,)), ((), ())),
                            preferred_element_type=jnp.float32)
    idx_ref[0, 0, :] = idx
    q_ref[0] = Q

    @pl.when(b == 0)
    def _init():
        counts_ref[...] = jnp.zeros_like(counts_ref)
        acc_ref[0] = 0.0

    Et = (idx[:, None]
          == jax.lax.broadcasted_iota(jnp.int32, (_HW, _K), 1))
    counts_ref[...] += jnp.sum(Et.astype(jnp.float32), axis=0)
    acc_ref[0] += jnp.sum(minv)

    @pl.when(b == _B - 1)
    def _finalize():
        loss_ref[...] = jnp.full((1, 1), acc_ref[0] * (1.25 / _NUMEL),
                                 jnp.float32)
        p = counts_ref[...] * (1.0 / _NTOK)
        perp_ref[...] = jnp.full((1, 1),
                                 jnp.exp(-jnp.sum(p * jnp.log(p + 1e-10))),
                                 jnp.float32)


def kernel(inputs, W):
    x3 = inputs.reshape(_B, _C, _HW)
    idx3, q3, loss11, perp11 = pl.pallas_call(
        _vq_body,
        grid=(_B,),
        in_specs=[pl.BlockSpec((1, _C, _HW), lambda b: (b, 0, 0)),
                  pl.BlockSpec((_K, _C), lambda b: (0, 0))],
        out_specs=[pl.BlockSpec((1, 1, _HW), lambda b: (b, 0, 0)),
                   pl.BlockSpec((1, _C, _HW), lambda b: (b, 0, 0)),
                   pl.BlockSpec((1, 1), lambda b: (0, 0)),
                   pl.BlockSpec((1, 1), lambda b: (0, 0))],
        out_shape=[jax.ShapeDtypeStruct((_B, 1, _HW), jnp.int32),
                   jax.ShapeDtypeStruct((_B, _C, _HW), jnp.float32),
                   jax.ShapeDtypeStruct((1, 1), jnp.float32),
                   jax.ShapeDtypeStruct((1, 1), jnp.float32)],
        scratch_shapes=[pltpu.VMEM((_K,), jnp.float32),
                        pltpu.SMEM((1,), jnp.float32)],
    )(x3, W)
    loss = loss11[0, 0]
    perplexity = perp11[0, 0]
    quantized_out = q3.reshape(inputs.shape)
    codebook_indices = idx3.reshape(-1)
    return (loss, quantized_out, perplexity, codebook_indices)


# speedup vs baseline: 1.8188x; 1.0745x over previous
import jax
import jax.numpy as jnp
from jax.experimental import pallas as pl
from jax.experimental.pallas import tpu as pltpu

_B = 16
_C = 64
_HW = 1024
_K = 1024
_NUMEL = _B * _C * _HW
_NTOK = _B * _HW


def _vq_body(x_ref, w_ref, idx_ref, q_ref, loss_ref, perp_ref, counts_ref,
             acc_ref, iota_ref, w16_ref, wn_ref):
    b = pl.program_id(0)
    X = x_ref[0]

    @pl.when(b == 0)
    def _init():
        W = w_ref[...]
        w16_ref[...] = W.astype(jnp.bfloat16) * jnp.bfloat16(2.0)
        wn_ref[...] = jnp.sum(W * W, axis=1, keepdims=True)
        iota_ref[...] = jax.lax.broadcasted_iota(
            jnp.int32, (_K, _HW), 0).astype(jnp.float32)
        counts_ref[...] = jnp.zeros_like(counts_ref)
        acc_ref[0] = 0.0

    W2_16 = w16_ref[...]
    M2 = jnp.dot(W2_16, X.astype(jnp.bfloat16),
                 preferred_element_type=jnp.float32)
    xn = jnp.sum(X * X, axis=0)
    T = (xn[None, :] + wn_ref[...]) - M2
    iota_kf = iota_ref[...]
    minv = jnp.min(T, axis=0)
    idxf = jnp.min(jnp.where(T == minv[None, :], iota_kf, float(_K)), axis=0)
    idx = idxf.astype(jnp.int32)
    E16 = jnp.where(iota_kf == idxf[None, :], 0.5, 0.0).astype(jnp.bfloat16)
    Q = jax.lax.dot_general(W2_16, E16, (((0,), (0,)), ((), ())),
                            preferred_element_type=jnp.float32)
    idx_ref[0, 0, :] = idx
    q_ref[0] = Q

    Et = (idx[:, None]
          == jax.lax.broadcasted_iota(jnp.int32, (_HW, _K), 1))
    counts_ref[...] += jnp.sum(Et.astype(jnp.float32), axis=0)
    acc_ref[0] += jnp.sum(minv)

    @pl.when(b == _B - 1)
    def _finalize():
        loss_ref[...] = jnp.full((1, 1), acc_ref[0] * (1.25 / _NUMEL),
                                 jnp.float32)
        p = counts_ref[...] * (1.0 / _NTOK)
        perp_ref[...] = jnp.full((1, 1),
                                 jnp.exp(-jnp.sum(p * jnp.log(p + 1e-10))),
                                 jnp.float32)


def kernel(inputs, W):
    x3 = inputs.reshape(_B, _C, _HW)
    idx3, q3, loss11, perp11 = pl.pallas_call(
        _vq_body,
        grid=(_B,),
        in_specs=[pl.BlockSpec((1, _C, _HW), lambda b: (b, 0, 0)),
                  pl.BlockSpec((_K, _C), lambda b: (0, 0))],
        out_specs=[pl.BlockSpec((1, 1, _HW), lambda b: (b, 0, 0)),
                   pl.BlockSpec((1, _C, _HW), lambda b: (b, 0, 0)),
                   pl.BlockSpec((1, 1), lambda b: (0, 0)),
                   pl.BlockSpec((1, 1), lambda b: (0, 0))],
        out_shape=[jax.ShapeDtypeStruct((_B, 1, _HW), jnp.int32),
                   jax.ShapeDtypeStruct((_B, _C, _HW), jnp.float32),
                   jax.ShapeDtypeStruct((1, 1), jnp.float32),
                   jax.ShapeDtypeStruct((1, 1), jnp.float32)],
        scratch_shapes=[pltpu.VMEM((_K,), jnp.float32),
                        pltpu.SMEM((1,), jnp.float32),
                        pltpu.VMEM((_K, _HW), jnp.float32),
                        pltpu.VMEM((_K, _C), jnp.bfloat16),
                        pltpu.VMEM((_K, 1), jnp.float32)],
    )(x3, W)
    loss = loss11[0, 0]
    perplexity = perp11[0, 0]
    quantized_out = q3.reshape(inputs.shape)
    codebook_indices = idx3.reshape(-1)
    return (loss, quantized_out, perplexity, codebook_indices)
